# Initial kernel scaffold; baseline (speedup 1.0000x reference)
import jax
import jax.numpy as jnp
from jax import lax
from jax.experimental import pallas as pl
from jax.experimental.pallas import tpu as pltpu
from jax.experimental.pallas import tpu_sc as plsc

_N = 10000
_E = 160000
_D = 256
_H = 128
_NS = 16
_NPAD = 10240

_EPT = _E // _NS
_B1 = 1000
_B2 = 400
_RPT = _N // _NS


def _sc_deg_dis(dst_hbm, ew_hbm, dis_hbm, acc_sh, zbuf, idxb, ewb, dbuf):
    c = lax.axis_index("c")
    s = lax.axis_index("s")

    @pl.loop(0, 40)
    def _zero(i):
        zbuf[pl.ds(i * 16, 16)] = jnp.zeros((16,), jnp.float32)

    pltpu.sync_copy(zbuf, acc_sh.at[pl.ds(s * 640, 640)])
    plsc.subcore_barrier()

    @pl.loop(0, _EPT // _B1)
    def _chunk(k):
        off = s * _EPT + k * _B1
        pltpu.sync_copy(dst_hbm.at[pl.ds(off, _B1)], idxb)
        pltpu.sync_copy(ew_hbm.at[pl.ds(off, _B1)], ewb)
        pltpu.sync_copy(ewb, acc_sh.at[idxb], add=True)

    plsc.subcore_barrier()
    pltpu.sync_copy(acc_sh.at[pl.ds(s * 640, 640)], dbuf)

    @pl.loop(0, 40)
    def _rsqrt(i):
        sl = pl.ds(i * 16, 16)
        v = dbuf[sl] + 1.0
        bits = plsc.bitcast(v, jnp.int32)
        y = plsc.bitcast(jnp.int32(0x5F3759DF) - (bits >> 1), jnp.float32)
        y = y * (1.5 - 0.5 * v * y * y)
        y = y * (1.5 - 0.5 * v * y * y)
        y = y * (1.5 - 0.5 * v * y * y)
        dbuf[sl] = y

    @pl.when(c == 0)
    def _write():
        pltpu.sync_copy(dbuf, dis_hbm.at[pl.ds(s * 640, 640)])


def _sc_message(y_hbm, src_hbm, dst_hbm, ew_hbm, out_hbm,
                acc_sh, zbuf, sidx, gidx, didx, ewb, rows, gsem):
    c = lax.axis_index("c")
    s = lax.axis_index("s")
    base = c * _N

    @pl.loop(0, 125)
    def _zr(r):
        @pl.loop(0, _H // 16)
        def _zc(j):
            zbuf[r, pl.ds(j * 16, 16)] = jnp.zeros((16,), jnp.float32)

    @pl.loop(0, _RPT // 125)
    def _zcopy(kk):
        pltpu.sync_copy(zbuf, acc_sh.at[pl.ds(s * _RPT + kk * 125, 125)])

    plsc.subcore_barrier()

    @pl.loop(0, _EPT // _B2)
    def _chunk(k):
        off = s * _EPT + k * _B2
        pltpu.sync_copy(src_hbm.at[pl.ds(off, _B2)], sidx)
        pltpu.sync_copy(dst_hbm.at[pl.ds(off, _B2)], didx)
        pltpu.sync_copy(ew_hbm.at[pl.ds(off, _B2)], ewb)

        @pl.loop(0, _B2 // 16)
        def _adj(i):
            sl = pl.ds(i * 16, 16)
            gidx[sl] = sidx[sl] + base

        pltpu.async_copy(y_hbm.at[gidx], rows, gsem).wait()

        @pl.loop(0, _B2)
        def _scale(e):
            w = ewb[e]
            for j in range(_H // 16):
                sl = pl.ds(j * 16, 16)
                rows[e, sl] = rows[e, sl] * w

        pltpu.sync_copy(rows, acc_sh.at[didx], add=True)

    plsc.subcore_barrier()
    pltpu.sync_copy(acc_sh.at[pl.ds(s * _RPT, _RPT)],
                    out_hbm.at[pl.ds(base + s * _RPT, _RPT)])


_MESH = plsc.VectorSubcoreMesh(core_axis_name="c", subcore_axis_name="s")

_deg_dis = pl.kernel(
    _sc_deg_dis,
    out_type=jax.ShapeDtypeStruct((_NPAD,), jnp.float32),
    mesh=_MESH,
    scratch_types=[
        pltpu.VMEM_SHARED((_NPAD,), jnp.float32),
        pltpu.VMEM((640,), jnp.float32),
        pltpu.VMEM((_B1,), jnp.int32),
        pltpu.VMEM((_B1,), jnp.float32),
        pltpu.VMEM((640,), jnp.float32),
    ],
)

_message = pl.kernel(
    _sc_message,
    out_type=jax.ShapeDtypeStruct((2 * _N, _H), jnp.float32),
    mesh=_MESH,
    scratch_types=[
        pltpu.VMEM_SHARED((_N, _H), jnp.float32),
        pltpu.VMEM((125, _H), jnp.float32),
        pltpu.VMEM((_B2,), jnp.int32),
        pltpu.VMEM((_B2,), jnp.int32),
        pltpu.VMEM((_B2,), jnp.int32),
        pltpu.VMEM((_B2,), jnp.float32),
        pltpu.VMEM((_B2, _H), jnp.float32),
        pltpu.SemaphoreType.DMA,
    ],
)


_BN = 1000
_G = _N // _BN


def _tc1_body(x_ref, w_ref, dis_ref, y_ref):
    xw = jnp.dot(x_ref[...], w_ref[...], preferred_element_type=jnp.float32)
    y_ref[...] = xw * dis_ref[...]


def _tc2_body(aa_ref, ab_ref, ya_ref, yb_ref, dis_ref, b1_ref, w2_ref, y2_ref):
    dis = dis_ref[...]
    b1 = b1_ref[...]
    za = dis * (aa_ref[...] + ya_ref[...]) + b1[:, :_H]
    zb = dis * (ab_ref[...] + yb_ref[...]) + b1[:, _H:]
    ha = jnp.where(za >= 0, za, 0.01 * za)
    hb = jnp.where(zb >= 0, zb, 0.01 * zb)
    w2 = w2_ref[...]
    y2 = (jnp.dot(ha, w2[:_H, :], preferred_element_type=jnp.float32)
          + jnp.dot(hb, w2[_H:, :], preferred_element_type=jnp.float32))
    y2_ref[...] = y2 * dis


def _tc3_body(aa_ref, ab_ref, ya_ref, yb_ref, dis_ref, b2_ref, out_ref):
    dis = dis_ref[...]
    oa = dis * (aa_ref[...] + ya_ref[...])
    ob = dis * (ab_ref[...] + yb_ref[...])
    out_ref[...] = jnp.concatenate([oa, ob], axis=1) + b2_ref[...]


def _tc1(x, w1, dis2d):
    return pl.pallas_call(
        _tc1_body,
        grid=(2, _G),
        in_specs=[
            pl.BlockSpec((_BN, _D), lambda c, i: (i, 0)),
            pl.BlockSpec((_D, _H), lambda c, i: (0, c)),
            pl.BlockSpec((_BN, 1), lambda c, i: (i, 0)),
        ],
        out_specs=pl.BlockSpec((_BN, _H), lambda c, i: (c * _G + i, 0)),
        out_shape=jax.ShapeDtypeStruct((2 * _N, _H), jnp.float32),
    )(x, w1, dis2d)


def _tc2(acc1, y1, dis2d, b1r, w2):
    half_a = pl.BlockSpec((_BN, _H), lambda c, i: (i, 0))
    half_b = pl.BlockSpec((_BN, _H), lambda c, i: (_G + i, 0))
    return pl.pallas_call(
        _tc2_body,
        grid=(2, _G),
        in_specs=[
            half_a, half_b, half_a, half_b,
            pl.BlockSpec((_BN, 1), lambda c, i: (i, 0)),
            pl.BlockSpec((1, _D), lambda c, i: (0, 0)),
            pl.BlockSpec((_D, _H), lambda c, i: (0, c)),
        ],
        out_specs=pl.BlockSpec((_BN, _H), lambda c, i: (c * _G + i, 0)),
        out_shape=jax.ShapeDtypeStruct((2 * _N, _H), jnp.float32),
    )(acc1, acc1, y1, y1, dis2d, b1r, w2)


def _tc3(acc2, y2, dis2d, b2r):
    half_a = pl.BlockSpec((_BN, _H), lambda i: (i, 0))
    half_b = pl.BlockSpec((_BN, _H), lambda i: (_G + i, 0))
    return pl.pallas_call(
        _tc3_body,
        grid=(_G,),
        in_specs=[
            half_a, half_b, half_a, half_b,
            pl.BlockSpec((_BN, 1), lambda i: (i, 0)),
            pl.BlockSpec((1, _D), lambda i: (0, 0)),
        ],
        out_specs=pl.BlockSpec((_BN, _D), lambda i: (i, 0)),
        out_shape=jax.ShapeDtypeStruct((_N, _D), jnp.float32),
    )(acc2, acc2, y2, y2, dis2d, b2r)


def kernel(x, edge_index, edge_weight, W1, b1, W2, b2):
    src = edge_index[0]
    dst = edge_index[1]
    dis_pad = _deg_dis(dst, edge_weight)
    dis2d = dis_pad[:_N].reshape(_N, 1)
    b1r = b1.reshape(1, _D)
    b2r = b2.reshape(1, _D)

    y1 = _tc1(x, W1, dis2d)
    acc1 = _message(y1, src, dst, edge_weight)
    y2 = _tc2(acc1, y1, dis2d, b1r, W2)
    acc2 = _message(y2, src, dst, edge_weight)
    return _tc3(acc2, y2, dis2d, b2r)


# speedup vs baseline: 5.6683x; 5.6683x over previous
import jax
import jax.numpy as jnp
from jax import lax
from jax.experimental import pallas as pl
from jax.experimental.pallas import tpu as pltpu
from jax.experimental.pallas import tpu_sc as plsc

_N = 10000
_E = 160000
_D = 256
_H = 128
_NS = 16
_NPAD = 10240

_EPT = _E // _NS
_B = 80


def _sc_deg(dst_hbm, ew_hbm, deg_hbm, acc_sh, zbuf, idxb, ewb, prow):
    c = lax.axis_index("c")
    s = lax.axis_index("s")

    @pl.loop(0, 640)
    def _zero(r):
        zbuf[r, pl.ds(0, 16)] = jnp.zeros((16,), jnp.float32)

    pltpu.sync_copy(zbuf, acc_sh.at[pl.ds(s * 640, 640)])
    plsc.subcore_barrier()

    @pl.loop(0, _EPT // _B)
    def _chunk(k):
        off = s * _EPT + k * _B
        pltpu.sync_copy(dst_hbm.at[pl.ds(off, _B)], idxb)
        pltpu.sync_copy(ew_hbm.at[pl.ds(off, _B)], ewb)

        @pl.loop(0, _B // 16)
        def _fill(g):
            wv = ewb[pl.ds(g * 16, 16)]
            for l in range(16):
                prow[g * 16 + l, pl.ds(0, 16)] = jnp.full((16,), wv[l],
                                                          jnp.float32)

        pltpu.sync_copy(prow, acc_sh.at[idxb], add=True)

    plsc.subcore_barrier()

    @pl.when(c == 0)
    def _write():
        pltpu.sync_copy(acc_sh.at[pl.ds(s * 640, 640)],
                        deg_hbm.at[pl.ds(s * 640, 640)])


def _sc_message(y_hbm, src_hbm, dst_hbm, ew_hbm, out_hbm,
                acc_sh, zbuf, sidx, gidx, didx, ewb, rows, gsem):
    c = lax.axis_index("c")
    s = lax.axis_index("s")
    base = c * _N

    @pl.when(s < 10)
    def _zinit():
        @pl.loop(0, 200)
        def _zr(r):
            @pl.loop(0, _H // 16)
            def _zc(j):
                zbuf[r, pl.ds(j * 16, 16)] = jnp.zeros((16,), jnp.float32)

        @pl.loop(0, 5)
        def _zcopy(kk):
            pltpu.sync_copy(zbuf, acc_sh.at[pl.ds(s * 1000 + kk * 200, 200)])

    plsc.subcore_barrier()

    @pl.loop(0, _EPT // _B)
    def _chunk(k):
        off = s * _EPT + k * _B
        pltpu.sync_copy(src_hbm.at[pl.ds(off, _B)], sidx)
        pltpu.sync_copy(dst_hbm.at[pl.ds(off, _B)], didx)
        pltpu.sync_copy(ew_hbm.at[pl.ds(off, _B)], ewb)

        @pl.loop(0, _B // 16)
        def _adj(i):
            sl = pl.ds(i * 16, 16)
            gidx[sl] = sidx[sl] + base

        pltpu.async_copy(y_hbm.at[gidx], rows, gsem).wait()

        @pl.loop(0, _B // 16)
        def _scale(g):
            wv = ewb[pl.ds(g * 16, 16)]
            for l in range(16):
                w = wv[l]
                e = g * 16 + l
                for j in range(_H // 16):
                    sl = pl.ds(j * 16, 16)
                    rows[e, sl] = rows[e, sl] * w

        pltpu.sync_copy(rows, acc_sh.at[didx], add=True)

    plsc.subcore_barrier()

    @pl.when(s < 10)
    def _copyout():
        pltpu.sync_copy(acc_sh.at[pl.ds(s * 1000, 1000)],
                        out_hbm.at[pl.ds(base + s * 1000, 1000)])


_MESH = plsc.VectorSubcoreMesh(core_axis_name="c", subcore_axis_name="s")

_deg = pl.kernel(
    _sc_deg,
    out_type=jax.ShapeDtypeStruct((_NPAD, 16), jnp.float32),
    mesh=_MESH,
    scratch_types=[
        pltpu.VMEM_SHARED((_NPAD, 16), jnp.float32),
        pltpu.VMEM((640, 16), jnp.float32),
        pltpu.VMEM((_B,), jnp.int32),
        pltpu.VMEM((_B,), jnp.float32),
        pltpu.VMEM((_B, 16), jnp.float32),
    ],
)

_message = pl.kernel(
    _sc_message,
    out_type=jax.ShapeDtypeStruct((2 * _N, _H), jnp.float32),
    mesh=_MESH,
    scratch_types=[
        pltpu.VMEM_SHARED((_N, _H), jnp.float32),
        pltpu.VMEM((200, _H), jnp.float32),
        pltpu.VMEM((_B,), jnp.int32),
        pltpu.VMEM((_B,), jnp.int32),
        pltpu.VMEM((_B,), jnp.int32),
        pltpu.VMEM((_B,), jnp.float32),
        pltpu.VMEM((_B, _H), jnp.float32),
        pltpu.SemaphoreType.DMA,
    ],
)


_BN = 1000
_G = _N // _BN


def _tc1_body(x_ref, w_ref, deg_ref, y_ref):
    dis = lax.rsqrt(1.0 + deg_ref[...][:, :1])
    xw = jnp.dot(x_ref[...], w_ref[...], preferred_element_type=jnp.float32)
    y_ref[...] = xw * dis


def _tc2_body(aa_ref, ab_ref, ya_ref, yb_ref, deg_ref, b1_ref, w2_ref, y2_ref):
    dis = lax.rsqrt(1.0 + deg_ref[...][:, :1])
    b1 = b1_ref[...]
    za = dis * (aa_ref[...] + ya_ref[...]) + b1[:, :_H]
    zb = dis * (ab_ref[...] + yb_ref[...]) + b1[:, _H:]
    ha = jnp.where(za >= 0, za, 0.01 * za)
    hb = jnp.where(zb >= 0, zb, 0.01 * zb)
    w2 = w2_ref[...]
    y2 = (jnp.dot(ha, w2[:_H, :], preferred_element_type=jnp.float32)
          + jnp.dot(hb, w2[_H:, :], preferred_element_type=jnp.float32))
    y2_ref[...] = y2 * dis


def _tc3_body(aa_ref, ab_ref, ya_ref, yb_ref, deg_ref, b2_ref, out_ref):
    dis = lax.rsqrt(1.0 + deg_ref[...][:, :1])
    oa = dis * (aa_ref[...] + ya_ref[...])
    ob = dis * (ab_ref[...] + yb_ref[...])
    out_ref[...] = jnp.concatenate([oa, ob], axis=1) + b2_ref[...]


def _tc1(x, w1, deg2d):
    return pl.pallas_call(
        _tc1_body,
        grid=(2, _G),
        in_specs=[
            pl.BlockSpec((_BN, _D), lambda c, i: (i, 0)),
            pl.BlockSpec((_D, _H), lambda c, i: (0, c)),
            pl.BlockSpec((_BN, 16), lambda c, i: (i, 0)),
        ],
        out_specs=pl.BlockSpec((_BN, _H), lambda c, i: (c * _G + i, 0)),
        out_shape=jax.ShapeDtypeStruct((2 * _N, _H), jnp.float32),
    )(x, w1, deg2d)


def _tc2(acc1, y1, deg2d, b1r, w2):
    half_a = pl.BlockSpec((_BN, _H), lambda c, i: (i, 0))
    half_b = pl.BlockSpec((_BN, _H), lambda c, i: (_G + i, 0))
    return pl.pallas_call(
        _tc2_body,
        grid=(2, _G),
        in_specs=[
            half_a, half_b, half_a, half_b,
            pl.BlockSpec((_BN, 16), lambda c, i: (i, 0)),
            pl.BlockSpec((1, _D), lambda c, i: (0, 0)),
            pl.BlockSpec((_D, _H), lambda c, i: (0, c)),
        ],
        out_specs=pl.BlockSpec((_BN, _H), lambda c, i: (c * _G + i, 0)),
        out_shape=jax.ShapeDtypeStruct((2 * _N, _H), jnp.float32),
    )(acc1, acc1, y1, y1, deg2d, b1r, w2)


def _tc3(acc2, y2, deg2d, b2r):
    half_a = pl.BlockSpec((_BN, _H), lambda i: (i, 0))
    half_b = pl.BlockSpec((_BN, _H), lambda i: (_G + i, 0))
    return pl.pallas_call(
        _tc3_body,
        grid=(_G,),
        in_specs=[
            half_a, half_b, half_a, half_b,
            pl.BlockSpec((_BN, 16), lambda i: (i, 0)),
            pl.BlockSpec((1, _D), lambda i: (0, 0)),
        ],
        out_specs=pl.BlockSpec((_BN, _D), lambda i: (i, 0)),
        out_shape=jax.ShapeDtypeStruct((_N, _D), jnp.float32),
    )(acc2, acc2, y2, y2, deg2d, b2r)


def kernel(x, edge_index, edge_weight, W1, b1, W2, b2):
    src = edge_index[0]
    dst = edge_index[1]
    deg2d = _deg(dst, edge_weight)[:_N]
    b1r = b1.reshape(1, _D)
    b2r = b2.reshape(1, _D)

    y1 = _tc1(x, W1, deg2d)

    acc1 = _message(y1, src, dst, edge_weight)
    y2 = _tc2(acc1, y1, deg2d, b1r, W2)
    acc2 = _message(y2, src, dst, edge_weight)
    return _tc3(acc2, y2, deg2d, b2r)


# speedup vs baseline: 11.0303x; 1.9460x over previous
import jax
import jax.numpy as jnp
from jax import lax
from jax.experimental import pallas as pl
from jax.experimental.pallas import tpu as pltpu
from jax.experimental.pallas import tpu_sc as plsc

_N = 10000
_E = 160000
_D = 256
_H = 128
_NS = 16
_NPAD = 10240

_EPT = _E // _NS
_B = 80
_PH = 5
_PC = _EPT // _PH // _B
_DB = _E // 2 // _NS
_DBC = 40


def _sc_deg(dst_hbm, ew_hbm, deg_hbm, acc_sh, zbuf, idxb, ewb, prow):
    c = lax.axis_index("c")
    s = lax.axis_index("s")

    @pl.loop(0, 640)
    def _zero(r):
        zbuf[r, pl.ds(0, 16)] = jnp.zeros((16,), jnp.float32)

    pltpu.sync_copy(zbuf, acc_sh.at[pl.ds(s * 640, 640)])
    plsc.subcore_barrier()

    @pl.loop(0, _DB // _DBC)
    def _chunk(k):
        off = c * (_E // 2) + s * _DB + k * _DBC
        pltpu.sync_copy(dst_hbm.at[pl.ds(off, _DBC)], idxb)
        pltpu.sync_copy(ew_hbm.at[pl.ds(off, _DBC)], ewb)

        @pl.loop(0, _DBC // 16)
        def _fill(g):
            wv = ewb[pl.ds(g * 16, 16)]
            for l in range(16):
                prow[g * 16 + l, pl.ds(0, 16)] = jnp.full((16,), wv[l],
                                                          jnp.float32)

        pltpu.sync_copy(prow, acc_sh.at[idxb], add=True)

    plsc.subcore_barrier()
    pltpu.sync_copy(acc_sh.at[pl.ds(s * 640, 640)],
                    deg_hbm.at[pl.ds(c * _NPAD + s * 640, 640)])


def _sc_message(y_hbm, src_hbm, dst_hbm, ew_hbm, out_hbm, acc_sh,
                gidx, didxs, ews, d80a, d80b, rows0, rows1,
                gsem0, gsem1, ssem0, ssem1):
    c = lax.axis_index("c")
    s = lax.axis_index("s")
    base = c * _N

    def scale(rows, k):
        @pl.loop(0, _B // 16)
        def _g(g):
            wv = ews[pl.ds(k * _B + g * 16, 16)]
            for l in range(16):
                w = wv[l]
                e = g * 16 + l
                for j in range(_H // 16):
                    sl = pl.ds(j * 16, 16)
                    rows[e, sl] = rows[e, sl] * w

    def fill(d80, k):
        @pl.loop(0, _B // 16)
        def _i(i):
            d80[pl.ds(i * 16, 16)] = didxs[pl.ds(k * _B + i * 16, 16)]

    def gather_start(k, rows, sem):
        pltpu.async_copy(y_hbm.at[gidx.at[pl.ds(k * _B, _B)]], rows, sem)

    def gather_wait(k, rows, sem):
        pltpu.make_async_copy(y_hbm.at[gidx.at[pl.ds(k * _B, _B)]],
                              rows, sem).wait()

    def scatter_start(rows, d80, sem):
        pltpu.async_copy(rows, acc_sh.at[d80], sem, add=True)

    def scatter_wait(rows, d80, sem):
        pltpu.make_async_copy(rows, acc_sh.at[d80], sem).wait()

    @pl.loop(0, _B)
    def _zr(r):
        for j in range(_H // 16):
            rows0[r, pl.ds(j * 16, 16)] = jnp.zeros((16,), jnp.float32)

    @pl.loop(0, _N // _B)
    def _zcopy(j):
        @pl.when(j % _NS == s)
        def _():
            pltpu.sync_copy(rows0, acc_sh.at[pl.ds(j * _B, _B)])

    plsc.subcore_barrier()

    @pl.loop(0, _PH)
    def _phase(p):
        poff = s * _EPT + p * (_PC * _B)
        pltpu.sync_copy(src_hbm.at[pl.ds(poff, _PC * _B)], gidx)
        pltpu.sync_copy(dst_hbm.at[pl.ds(poff, _PC * _B)], didxs)
        pltpu.sync_copy(ew_hbm.at[pl.ds(poff, _PC * _B)], ews)

        @pl.loop(0, _PC * _B // 16)
        def _adj(i):
            sl = pl.ds(i * 16, 16)
            gidx[sl] = gidx[sl] + base

        gather_start(0, rows0, gsem0)

        @pl.loop(0, (_PC - 1) // 2)
        def _pair(t):
            e = 2 * t
            o = 2 * t + 1
            gather_start(o, rows1, gsem1)
            gather_wait(e, rows0, gsem0)
            scale(rows0, e)
            fill(d80a, e)
            scatter_start(rows0, d80a, ssem0)
            scatter_wait(rows0, d80a, ssem0)
            gather_start(o + 1, rows0, gsem0)
            gather_wait(o, rows1, gsem1)
            scale(rows1, o)
            fill(d80b, o)
            scatter_start(rows1, d80b, ssem1)
            scatter_wait(rows1, d80b, ssem1)

        last = _PC - 1
        gather_wait(last, rows0, gsem0)
        scale(rows0, last)
        fill(d80a, last)
        scatter_start(rows0, d80a, ssem0)
        scatter_wait(rows0, d80a, ssem0)

    plsc.subcore_barrier()

    @pl.when(s < 10)
    def _copyout():
        pltpu.sync_copy(acc_sh.at[pl.ds(s * 1000, 1000)],
                        out_hbm.at[pl.ds(base + s * 1000, 1000)])


_MESH = plsc.VectorSubcoreMesh(core_axis_name="c", subcore_axis_name="s")

_deg = pl.kernel(
    _sc_deg,
    out_type=jax.ShapeDtypeStruct((2 * _NPAD, 16), jnp.float32),
    mesh=_MESH,
    scratch_types=[
        pltpu.VMEM_SHARED((_NPAD, 16), jnp.float32),
        pltpu.VMEM((640, 16), jnp.float32),
        pltpu.VMEM((_DBC,), jnp.int32),
        pltpu.VMEM((_DBC,), jnp.float32),
        pltpu.VMEM((_DBC, 16), jnp.float32),
    ],
)

_message = pl.kernel(
    _sc_message,
    out_type=jax.ShapeDtypeStruct((2 * _N, _H), jnp.float32),
    mesh=_MESH,
    scratch_types=[
        pltpu.VMEM_SHARED((_N, _H), jnp.float32),
        pltpu.VMEM((_PC * _B,), jnp.int32),
        pltpu.VMEM((_PC * _B,), jnp.int32),
        pltpu.VMEM((_PC * _B,), jnp.float32),
        pltpu.VMEM((_B,), jnp.int32),
        pltpu.VMEM((_B,), jnp.int32),
        pltpu.VMEM((_B, _H), jnp.float32),
        pltpu.VMEM((_B, _H), jnp.float32),
        pltpu.SemaphoreType.DMA,
        pltpu.SemaphoreType.DMA,
        pltpu.SemaphoreType.DMA,
        pltpu.SemaphoreType.DMA,
    ],
)


_BN = 1000
_G = _N // _BN


def _tc1_body(x_ref, w_ref, dega_ref, degb_ref, y_ref):
    dis = lax.rsqrt(1.0 + (dega_ref[...] + degb_ref[...])[:, :1])
    xw = jnp.dot(x_ref[...], w_ref[...], preferred_element_type=jnp.float32)
    y_ref[...] = xw * dis


def _tc2_body(aa_ref, ab_ref, ya_ref, yb_ref, dega_ref, degb_ref,
              b1_ref, w2_ref, y2_ref):
    dis = lax.rsqrt(1.0 + (dega_ref[...] + degb_ref[...])[:, :1])
    b1 = b1_ref[...]
    za = dis * (aa_ref[...] + ya_ref[...]) + b1[:, :_H]
    zb = dis * (ab_ref[...] + yb_ref[...]) + b1[:, _H:]
    ha = jnp.where(za >= 0, za, 0.01 * za)
    hb = jnp.where(zb >= 0, zb, 0.01 * zb)
    w2 = w2_ref[...]
    y2 = (jnp.dot(ha, w2[:_H, :], preferred_element_type=jnp.float32)
          + jnp.dot(hb, w2[_H:, :], preferred_element_type=jnp.float32))
    y2_ref[...] = y2 * dis


def _tc3_body(aa_ref, ab_ref, ya_ref, yb_ref, dega_ref, degb_ref,
              b2_ref, out_ref):
    dis = lax.rsqrt(1.0 + (dega_ref[...] + degb_ref[...])[:, :1])
    oa = dis * (aa_ref[...] + ya_ref[...])
    ob = dis * (ab_ref[...] + yb_ref[...])
    out_ref[...] = jnp.concatenate([oa, ob], axis=1) + b2_ref[...]


def _tc1(x, w1, dega, degb):
    deg_spec = pl.BlockSpec((_BN, 16), lambda c, i: (i, 0))
    return pl.pallas_call(
        _tc1_body,
        grid=(2, _G),
        in_specs=[
            pl.BlockSpec((_BN, _D), lambda c, i: (i, 0)),
            pl.BlockSpec((_D, _H), lambda c, i: (0, c)),
            deg_spec, deg_spec,
        ],
        out_specs=pl.BlockSpec((_BN, _H), lambda c, i: (c * _G + i, 0)),
        out_shape=jax.ShapeDtypeStruct((2 * _N, _H), jnp.float32),
    )(x, w1, dega, degb)


def _tc2(acc1, y1, dega, degb, b1r, w2):
    half_a = pl.BlockSpec((_BN, _H), lambda c, i: (i, 0))
    half_b = pl.BlockSpec((_BN, _H), lambda c, i: (_G + i, 0))
    deg_spec = pl.BlockSpec((_BN, 16), lambda c, i: (i, 0))
    return pl.pallas_call(
        _tc2_body,
        grid=(2, _G),
        in_specs=[
            half_a, half_b, half_a, half_b,
            deg_spec, deg_spec,
            pl.BlockSpec((1, _D), lambda c, i: (0, 0)),
            pl.BlockSpec((_D, _H), lambda c, i: (0, c)),
        ],
        out_specs=pl.BlockSpec((_BN, _H), lambda c, i: (c * _G + i, 0)),
        out_shape=jax.ShapeDtypeStruct((2 * _N, _H), jnp.float32),
    )(acc1, acc1, y1, y1, dega, degb, b1r, w2)


def _tc3(acc2, y2, dega, degb, b2r):
    half_a = pl.BlockSpec((_BN, _H), lambda i: (i, 0))
    half_b = pl.BlockSpec((_BN, _H), lambda i: (_G + i, 0))
    deg_spec = pl.BlockSpec((_BN, 16), lambda i: (i, 0))
    return pl.pallas_call(
        _tc3_body,
        grid=(_G,),
        in_specs=[
            half_a, half_b, half_a, half_b,
            deg_spec, deg_spec,
            pl.BlockSpec((1, _D), lambda i: (0, 0)),
        ],
        out_specs=pl.BlockSpec((_BN, _D), lambda i: (i, 0)),
        out_shape=jax.ShapeDtypeStruct((_N, _D), jnp.float32),
    )(acc2, acc2, y2, y2, dega, degb, b2r)


def kernel(x, edge_index, edge_weight, W1, b1, W2, b2):
    src = edge_index[0]
    dst = edge_index[1]
    deg_pad = _deg(dst, edge_weight)
    dega = deg_pad[:_N]
    degb = deg_pad[_NPAD:_NPAD + _N]
    b1r = b1.reshape(1, _D)
    b2r = b2.reshape(1, _D)

    y1 = _tc1(x, W1, dega, degb)
    acc1 = _message(y1, src, dst, edge_weight)
    y2 = _tc2(acc1, y1, dega, degb, b1r, W2)
    acc2 = _message(y2, src, dst, edge_weight)
    return _tc3(acc2, y2, dega, degb, b2r)


# speedup vs baseline: 14.0506x; 1.2738x over previous
import jax
import jax.numpy as jnp
from jax import lax
from jax.experimental import pallas as pl
from jax.experimental.pallas import tpu as pltpu
from jax.experimental.pallas import tpu_sc as plsc

_N = 10000
_E = 160000
_D = 256
_H = 128
_NS = 16
_NPAD = 10240

_EPT = _E // _NS
_B = 80
_PH = 5
_PC = _EPT // _PH // _B
_DB = _E // 2 // _NS
_DBC = 128


def _sc_deg(dst_hbm, ew_hbm, deg_hbm, acc_sh, zbuf, idxb, ewb, prow,
            idx8, ew16):
    c = lax.axis_index("c")
    s = lax.axis_index("s")

    @pl.loop(0, 640)
    def _zero(r):
        zbuf[r, pl.ds(0, 16)] = jnp.zeros((16,), jnp.float32)

    pltpu.sync_copy(zbuf, acc_sh.at[pl.ds(s * 640, 640)])
    plsc.subcore_barrier()

    @pl.loop(0, _DB // _DBC)
    def _chunk(k):
        off = c * (_E // 2) + s * _DB + k * _DBC
        pltpu.sync_copy(dst_hbm.at[pl.ds(off, _DBC)], idxb)
        pltpu.sync_copy(ew_hbm.at[pl.ds(off, _DBC)], ewb)

        @pl.loop(0, _DBC // 16)
        def _fill(g):
            wv = ewb[pl.ds(g * 16, 16)]
            for l in range(16):
                prow[g * 16 + l, pl.ds(0, 16)] = jnp.full((16,), wv[l],
                                                          jnp.float32)

        pltpu.sync_copy(prow, acc_sh.at[idxb], add=True)

    toff = c * (_E // 2) + s * _DB + (_DB // _DBC) * _DBC
    pltpu.sync_copy(dst_hbm.at[pl.ds(toff, 8)], idx8)
    pltpu.sync_copy(ew_hbm.at[pl.ds(toff, 8)], ew16.at[pl.ds(0, 8)])
    wv8 = ew16[pl.ds(0, 16)]
    for l in range(8):
        prow[l, pl.ds(0, 16)] = jnp.full((16,), wv8[l], jnp.float32)
    pltpu.sync_copy(prow.at[pl.ds(0, 8)], acc_sh.at[idx8], add=True)

    plsc.subcore_barrier()
    pltpu.sync_copy(acc_sh.at[pl.ds(s * 640, 640)],
                    deg_hbm.at[pl.ds(c * _NPAD + s * 640, 640)])


def _sc_message(y_hbm, src_hbm, dst_hbm, ew_hbm, out_hbm, acc_sh,
                gidx, didxs, ews, d0, d1, d2, d3, r0, r1, r2, r3,
                g0, g1, g2, g3, s0, s1, s2, s3):
    c = lax.axis_index("c")
    s = lax.axis_index("s")
    base = c * _N
    rows = (r0, r1, r2, r3)
    d80 = (d0, d1, d2, d3)
    gsem = (g0, g1, g2, g3)
    ssem = (s0, s1, s2, s3)

    def scale(rbuf, k):
        @pl.loop(0, _B // 16)
        def _g(g):
            wv = ews[pl.ds(k * _B + g * 16, 16)]
            for l in range(16):
                w = wv[l]
                e = g * 16 + l
                for j in range(_H // 16):
                    sl = pl.ds(j * 16, 16)
                    rbuf[e, sl] = rbuf[e, sl] * w

    def fill(dbuf, k):
        @pl.loop(0, _B // 16)
        def _i(i):
            dbuf[pl.ds(i * 16, 16)] = didxs[pl.ds(k * _B + i * 16, 16)]

    def gather_start(k, i):
        pltpu.async_copy(y_hbm.at[gidx.at[pl.ds(k * _B, _B)]], rows[i],
                         gsem[i])

    def gather_wait(k, i):
        pltpu.make_async_copy(y_hbm.at[gidx.at[pl.ds(k * _B, _B)]],
                              rows[i], gsem[i]).wait()

    def scatter_start(i):
        pltpu.async_copy(rows[i], acc_sh.at[d80[i]], ssem[i], add=True)

    def scatter_wait(i):
        pltpu.make_async_copy(rows[i], acc_sh.at[d80[i]], ssem[i]).wait()

    @pl.loop(0, _B)
    def _zr(r):
        for j in range(_H // 16):
            r0[r, pl.ds(j * 16, 16)] = jnp.zeros((16,), jnp.float32)

    @pl.loop(0, _N // _B)
    def _zcopy(j):
        @pl.when(j % _NS == s)
        def _():
            pltpu.sync_copy(r0, acc_sh.at[pl.ds(j * _B, _B)])

    plsc.subcore_barrier()

    @pl.loop(0, _PH)
    def _phase(p):
        poff = s * _EPT + p * (_PC * _B)
        pltpu.sync_copy(src_hbm.at[pl.ds(poff, _PC * _B)], gidx)
        pltpu.sync_copy(dst_hbm.at[pl.ds(poff, _PC * _B)], didxs)
        pltpu.sync_copy(ew_hbm.at[pl.ds(poff, _PC * _B)], ews)

        @pl.loop(0, _PC * _B // 16)
        def _adj(i):
            sl = pl.ds(i * 16, 16)
            gidx[sl] = gidx[sl] + base

        gather_start(0, 0)
        gather_start(1, 1)
        gather_start(2, 2)

        @pl.loop(0, _PC // 4)
        def _quad(t):
            k0 = 4 * t
            gather_wait(k0, 0)
            scale(r0, k0)
            fill(d0, k0)
            scatter_start(0)

            @pl.when(t >= 1)
            def _w0():
                scatter_wait(3)
            gather_start(k0 + 3, 3)
            gather_wait(k0 + 1, 1)
            scale(r1, k0 + 1)
            fill(d1, k0 + 1)
            scatter_start(1)
            scatter_wait(0)
            gather_start(k0 + 4, 0)
            gather_wait(k0 + 2, 2)
            scale(r2, k0 + 2)
            fill(d2, k0 + 2)
            scatter_start(2)

            @pl.when(t < _PC // 4 - 1)
            def _i2():
                scatter_wait(1)
                gather_start(k0 + 5, 1)
            gather_wait(k0 + 3, 3)
            scale(r3, k0 + 3)
            fill(d3, k0 + 3)
            scatter_start(3)

            @pl.when(t < _PC // 4 - 1)
            def _i3():
                scatter_wait(2)
                gather_start(k0 + 6, 2)

        last = _PC - 1
        gather_wait(last, 0)
        scale(r0, last)
        fill(d0, last)
        scatter_start(0)
        scatter_wait(1)
        scatter_wait(2)
        scatter_wait(3)
        scatter_wait(0)

    plsc.subcore_barrier()

    @pl.when(s < 10)
    def _copyout():
        pltpu.sync_copy(acc_sh.at[pl.ds(s * 1000, 1000)],
                        out_hbm.at[pl.ds(base + s * 1000, 1000)])


_MESH = plsc.VectorSubcoreMesh(core_axis_name="c", subcore_axis_name="s")

_deg = pl.kernel(
    _sc_deg,
    out_type=jax.ShapeDtypeStruct((2 * _NPAD, 16), jnp.float32),
    mesh=_MESH,
    scratch_types=[
        pltpu.VMEM_SHARED((_NPAD, 16), jnp.float32),
        pltpu.VMEM((640, 16), jnp.float32),
        pltpu.VMEM((_DBC,), jnp.int32),
        pltpu.VMEM((_DBC,), jnp.float32),
        pltpu.VMEM((_DBC, 16), jnp.float32),
        pltpu.VMEM((8,), jnp.int32),
        pltpu.VMEM((16,), jnp.float32),
    ],
)

_message = pl.kernel(
    _sc_message,
    out_type=jax.ShapeDtypeStruct((2 * _N, _H), jnp.float32),
    mesh=_MESH,
    scratch_types=[
        pltpu.VMEM_SHARED((_N, _H), jnp.float32),
        pltpu.VMEM((_PC * _B,), jnp.int32),
        pltpu.VMEM((_PC * _B,), jnp.int32),
        pltpu.VMEM((_PC * _B,), jnp.float32),
        pltpu.VMEM((_B,), jnp.int32),
        pltpu.VMEM((_B,), jnp.int32),
        pltpu.VMEM((_B,), jnp.int32),
        pltpu.VMEM((_B,), jnp.int32),
        pltpu.VMEM((_B, _H), jnp.float32),
        pltpu.VMEM((_B, _H), jnp.float32),
        pltpu.VMEM((_B, _H), jnp.float32),
        pltpu.VMEM((_B, _H), jnp.float32),
        pltpu.SemaphoreType.DMA,
        pltpu.SemaphoreType.DMA,
        pltpu.SemaphoreType.DMA,
        pltpu.SemaphoreType.DMA,
        pltpu.SemaphoreType.DMA,
        pltpu.SemaphoreType.DMA,
        pltpu.SemaphoreType.DMA,
        pltpu.SemaphoreType.DMA,
    ],
)


_BN = 1000
_G = _N // _BN


def _tc1_body(x_ref, w_ref, dega_ref, degb_ref, y_ref):
    dis = lax.rsqrt(1.0 + (dega_ref[...] + degb_ref[...])[:, :1])
    xw = jnp.dot(x_ref[...], w_ref[...], preferred_element_type=jnp.float32)
    y_ref[...] = xw * dis


def _tc2_body(aa_ref, ab_ref, ya_ref, yb_ref, dega_ref, degb_ref,
              b1_ref, w2_ref, y2_ref):
    dis = lax.rsqrt(1.0 + (dega_ref[...] + degb_ref[...])[:, :1])
    b1 = b1_ref[...]
    za = dis * (aa_ref[...] + ya_ref[...]) + b1[:, :_H]
    zb = dis * (ab_ref[...] + yb_ref[...]) + b1[:, _H:]
    ha = jnp.where(za >= 0, za, 0.01 * za)
    hb = jnp.where(zb >= 0, zb, 0.01 * zb)
    w2 = w2_ref[...]
    y2 = (jnp.dot(ha, w2[:_H, :], preferred_element_type=jnp.float32)
          + jnp.dot(hb, w2[_H:, :], preferred_element_type=jnp.float32))
    y2_ref[...] = y2 * dis


def _tc3_body(aa_ref, ab_ref, ya_ref, yb_ref, dega_ref, degb_ref,
              b2_ref, out_ref):
    dis = lax.rsqrt(1.0 + (dega_ref[...] + degb_ref[...])[:, :1])
    oa = dis * (aa_ref[...] + ya_ref[...])
    ob = dis * (ab_ref[...] + yb_ref[...])
    out_ref[...] = jnp.concatenate([oa, ob], axis=1) + b2_ref[...]


def _tc1(x, w1, dega, degb):
    deg_spec = pl.BlockSpec((_BN, 16), lambda c, i: (i, 0))
    return pl.pallas_call(
        _tc1_body,
        grid=(2, _G),
        in_specs=[
            pl.BlockSpec((_BN, _D), lambda c, i: (i, 0)),
            pl.BlockSpec((_D, _H), lambda c, i: (0, c)),
            deg_spec, deg_spec,
        ],
        out_specs=pl.BlockSpec((_BN, _H), lambda c, i: (c * _G + i, 0)),
        out_shape=jax.ShapeDtypeStruct((2 * _N, _H), jnp.float32),
    )(x, w1, dega, degb)


def _tc2(acc1, y1, dega, degb, b1r, w2):
    half_a = pl.BlockSpec((_BN, _H), lambda c, i: (i, 0))
    half_b = pl.BlockSpec((_BN, _H), lambda c, i: (_G + i, 0))
    deg_spec = pl.BlockSpec((_BN, 16), lambda c, i: (i, 0))
    return pl.pallas_call(
        _tc2_body,
        grid=(2, _G),
        in_specs=[
            half_a, half_b, half_a, half_b,
            deg_spec, deg_spec,
            pl.BlockSpec((1, _D), lambda c, i: (0, 0)),
            pl.BlockSpec((_D, _H), lambda c, i: (0, c)),
        ],
        out_specs=pl.BlockSpec((_BN, _H), lambda c, i: (c * _G + i, 0)),
        out_shape=jax.ShapeDtypeStruct((2 * _N, _H), jnp.float32),
    )(acc1, acc1, y1, y1, dega, degb, b1r, w2)


def _tc3(acc2, y2, dega, degb, b2r):
    half_a = pl.BlockSpec((_BN, _H), lambda i: (i, 0))
    half_b = pl.BlockSpec((_BN, _H), lambda i: (_G + i, 0))
    deg_spec = pl.BlockSpec((_BN, 16), lambda i: (i, 0))
    return pl.pallas_call(
        _tc3_body,
        grid=(_G,),
        in_specs=[
            half_a, half_b, half_a, half_b,
            deg_spec, deg_spec,
            pl.BlockSpec((1, _D), lambda i: (0, 0)),
        ],
        out_specs=pl.BlockSpec((_BN, _D), lambda i: (i, 0)),
        out_shape=jax.ShapeDtypeStruct((_N, _D), jnp.float32),
    )(acc2, acc2, y2, y2, dega, degb, b2r)


def kernel(x, edge_index, edge_weight, W1, b1, W2, b2):
    src = edge_index[0]
    dst = edge_index[1]
    deg_pad = _deg(dst, edge_weight)
    dega = deg_pad[:_N]
    degb = deg_pad[_NPAD:_NPAD + _N]
    b1r = b1.reshape(1, _D)
    b2r = b2.reshape(1, _D)

    y1 = _tc1(x, W1, dega, degb)
    acc1 = _message(y1, src, dst, edge_weight)
    y2 = _tc2(acc1, y1, dega, degb, b1r, W2)
    acc2 = _message(y2, src, dst, edge_weight)
    return _tc3(acc2, y2, dega, degb, b2r)


# speedup vs baseline: 15.7871x; 1.1236x over previous
import jax
import jax.numpy as jnp
from jax import lax
from jax.experimental import pallas as pl
from jax.experimental.pallas import tpu as pltpu
from jax.experimental.pallas import tpu_sc as plsc

_N = 10000
_E = 160000
_D = 256
_H = 128
_NS = 16
_NPAD = 10240

_EPT = _E // _NS
_B = 80
_PH = 5
_PC = _EPT // _PH // _B
_DB = _E // 2 // _NS
_DBC = 128


def _sc_deg(dst_hbm, ew_hbm, deg_hbm, acc_sh, zbuf,
            ib0, ib1, eb0, eb1, dx0, dx1, pr0, pr1,
            idx8, ew16, lsem0, lsem1, scs0, scs1):
    c = lax.axis_index("c")
    s = lax.axis_index("s")
    ibs = (ib0, ib1)
    ebs = (eb0, eb1)
    dxs = (dx0, dx1)
    prs = (pr0, pr1)
    lsem = (lsem0, lsem1)
    scs = (scs0, scs1)
    base_off = c * (_E // 2) + s * _DB

    def loads_start(k, i):
        pltpu.async_copy(dst_hbm.at[pl.ds(base_off + k * _DBC, _DBC)],
                         ibs[i], lsem[i])
        pltpu.async_copy(ew_hbm.at[pl.ds(base_off + k * _DBC, _DBC)],
                         ebs[i], lsem[i])

    def loads_wait(k, i):
        pltpu.make_async_copy(dst_hbm.at[pl.ds(base_off + k * _DBC, _DBC)],
                              ibs[i], lsem[i]).wait()
        pltpu.make_async_copy(ew_hbm.at[pl.ds(base_off + k * _DBC, _DBC)],
                              ebs[i], lsem[i]).wait()

    def prep(i):
        @pl.loop(0, _DBC // 16)
        def _cp(g):
            sl = pl.ds(g * 16, 16)
            dxs[i][sl] = ibs[i][sl]

        @pl.loop(0, _DBC // 16)
        def _fill(g):
            wv = ebs[i][pl.ds(g * 16, 16)]
            for l in range(16):
                prs[i][g * 16 + l, pl.ds(0, 16)] = jnp.full((16,), wv[l],
                                                            jnp.float32)

    def scatter_start(i):
        pltpu.async_copy(prs[i], acc_sh.at[dxs[i]], scs[i], add=True)

    def scatter_wait(i):
        pltpu.make_async_copy(prs[i], acc_sh.at[dxs[i]], scs[i]).wait()

    @pl.loop(0, 640)
    def _zero(r):
        zbuf[r, pl.ds(0, 16)] = jnp.zeros((16,), jnp.float32)

    pltpu.sync_copy(zbuf, acc_sh.at[pl.ds(s * 640, 640)])
    plsc.subcore_barrier()

    loads_start(0, 0)
    loads_start(1, 1)

    @pl.loop(0, 19)
    def _pair(t):
        e = 2 * t
        o = 2 * t + 1
        loads_wait(e, 0)

        @pl.when(t >= 1)
        def _w0():
            scatter_wait(0)
        prep(0)
        loads_start(e + 2, 0)
        scatter_start(0)
        loads_wait(o, 1)

        @pl.when(t >= 1)
        def _w1():
            scatter_wait(1)
        prep(1)

        @pl.when(t < 18)
        def _l1():
            loads_start(o + 2, 1)
        scatter_start(1)

    loads_wait(38, 0)
    scatter_wait(0)
    prep(0)
    scatter_start(0)

    toff = base_off + 39 * _DBC
    pltpu.sync_copy(dst_hbm.at[pl.ds(toff, 8)], idx8)
    pltpu.sync_copy(ew_hbm.at[pl.ds(toff, 8)], ew16.at[pl.ds(0, 8)])
    scatter_wait(1)
    wv8 = ew16[pl.ds(0, 16)]
    for l in range(8):
        pr1[l, pl.ds(0, 16)] = jnp.full((16,), wv8[l], jnp.float32)
    pltpu.sync_copy(pr1.at[pl.ds(0, 8)], acc_sh.at[idx8], add=True)
    scatter_wait(0)

    plsc.subcore_barrier()
    pltpu.sync_copy(acc_sh.at[pl.ds(s * 640, 640)],
                    deg_hbm.at[pl.ds(c * _NPAD + s * 640, 640)])


def _sc_message(y_hbm, src_hbm, dst_hbm, ew_hbm, out_hbm, acc_sh,
                gidx, didxs, ews, d0, d1, d2, d3, r0, r1, r2, r3,
                g0, g1, g2, g3, s0, s1, s2, s3, stsem):
    c = lax.axis_index("c")
    s = lax.axis_index("s")
    base = c * _N
    rows = (r0, r1, r2, r3)
    d80 = (d0, d1, d2, d3)
    gsem = (g0, g1, g2, g3)
    ssem = (s0, s1, s2, s3)

    def scale(rbuf, k):
        @pl.loop(0, _B // 16)
        def _g(g):
            wv = ews[pl.ds(k * _B + g * 16, 16)]
            for l in range(16):
                wb = jnp.full((16,), wv[l], jnp.float32)
                e = g * 16 + l
                for j in range(_H // 16):
                    sl = pl.ds(j * 16, 16)
                    rbuf[e, sl] = rbuf[e, sl] * wb

    def fill(dbuf, k):
        @pl.loop(0, _B // 16)
        def _i(i):
            dbuf[pl.ds(i * 16, 16)] = didxs[pl.ds(k * _B + i * 16, 16)]

    def gather_start(k, i):
        pltpu.async_copy(y_hbm.at[gidx.at[pl.ds(k * _B, _B)]], rows[i],
                         gsem[i])

    def gather_wait(k, i):
        pltpu.make_async_copy(y_hbm.at[gidx.at[pl.ds(k * _B, _B)]],
                              rows[i], gsem[i]).wait()

    def scatter_start(i):
        pltpu.async_copy(rows[i], acc_sh.at[d80[i]], ssem[i], add=True)

    def scatter_wait(i):
        pltpu.make_async_copy(rows[i], acc_sh.at[d80[i]], ssem[i]).wait()

    @pl.loop(0, _B)
    def _zr(r):
        for j in range(_H // 16):
            r0[r, pl.ds(j * 16, 16)] = jnp.zeros((16,), jnp.float32)

    @pl.loop(0, _N // _B)
    def _zcopy(j):
        @pl.when(j % _NS == s)
        def _():
            pltpu.sync_copy(r0, acc_sh.at[pl.ds(j * _B, _B)])

    plsc.subcore_barrier()

    @pl.loop(0, _PH)
    def _phase(p):
        poff = s * _EPT + p * (_PC * _B)
        st1 = pltpu.async_copy(src_hbm.at[pl.ds(poff, _PC * _B)], gidx, stsem)
        st2 = pltpu.async_copy(dst_hbm.at[pl.ds(poff, _PC * _B)], didxs, stsem)
        st3 = pltpu.async_copy(ew_hbm.at[pl.ds(poff, _PC * _B)], ews, stsem)
        st1.wait()
        st2.wait()
        st3.wait()

        @pl.loop(0, _PC * _B // 16)
        def _adj(i):
            sl = pl.ds(i * 16, 16)
            gidx[sl] = gidx[sl] + base

        gather_start(0, 0)
        gather_start(1, 1)
        gather_start(2, 2)

        @pl.loop(0, _PC // 4)
        def _quad(t):
            k0 = 4 * t
            gather_wait(k0, 0)
            scale(r0, k0)
            fill(d0, k0)
            scatter_start(0)

            @pl.when(t >= 1)
            def _w0():
                scatter_wait(3)
            gather_start(k0 + 3, 3)
            gather_wait(k0 + 1, 1)
            scale(r1, k0 + 1)
            fill(d1, k0 + 1)
            scatter_start(1)
            scatter_wait(0)
            gather_start(k0 + 4, 0)
            gather_wait(k0 + 2, 2)
            scale(r2, k0 + 2)
            fill(d2, k0 + 2)
            scatter_start(2)

            @pl.when(t < _PC // 4 - 1)
            def _i2():
                scatter_wait(1)
                gather_start(k0 + 5, 1)
            gather_wait(k0 + 3, 3)
            scale(r3, k0 + 3)
            fill(d3, k0 + 3)
            scatter_start(3)

            @pl.when(t < _PC // 4 - 1)
            def _i3():
                scatter_wait(2)
                gather_start(k0 + 6, 2)

        last = _PC - 1
        gather_wait(last, 0)
        scale(r0, last)
        fill(d0, last)
        scatter_start(0)
        scatter_wait(1)
        scatter_wait(2)
        scatter_wait(3)
        scatter_wait(0)

    plsc.subcore_barrier()

    @pl.when(s < 10)
    def _copyout():
        pltpu.sync_copy(acc_sh.at[pl.ds(s * 1000, 1000)],
                        out_hbm.at[pl.ds(base + s * 1000, 1000)])


_MESH = plsc.VectorSubcoreMesh(core_axis_name="c", subcore_axis_name="s")

_deg = pl.kernel(
    _sc_deg,
    out_type=jax.ShapeDtypeStruct((2 * _NPAD, 16), jnp.float32),
    mesh=_MESH,
    scratch_types=[
        pltpu.VMEM_SHARED((_NPAD, 16), jnp.float32),
        pltpu.VMEM((640, 16), jnp.float32),
        pltpu.VMEM((_DBC,), jnp.int32),
        pltpu.VMEM((_DBC,), jnp.int32),
        pltpu.VMEM((_DBC,), jnp.float32),
        pltpu.VMEM((_DBC,), jnp.float32),
        pltpu.VMEM((_DBC,), jnp.int32),
        pltpu.VMEM((_DBC,), jnp.int32),
        pltpu.VMEM((_DBC, 16), jnp.float32),
        pltpu.VMEM((_DBC, 16), jnp.float32),
        pltpu.VMEM((8,), jnp.int32),
        pltpu.VMEM((16,), jnp.float32),
        pltpu.SemaphoreType.DMA,
        pltpu.SemaphoreType.DMA,
        pltpu.SemaphoreType.DMA,
        pltpu.SemaphoreType.DMA,
    ],
)

_message = pl.kernel(
    _sc_message,
    out_type=jax.ShapeDtypeStruct((2 * _N, _H), jnp.float32),
    mesh=_MESH,
    scratch_types=[
        pltpu.VMEM_SHARED((_N, _H), jnp.float32),
        pltpu.VMEM((_PC * _B,), jnp.int32),
        pltpu.VMEM((_PC * _B,), jnp.int32),
        pltpu.VMEM((_PC * _B,), jnp.float32),
        pltpu.VMEM((_B,), jnp.int32),
        pltpu.VMEM((_B,), jnp.int32),
        pltpu.VMEM((_B,), jnp.int32),
        pltpu.VMEM((_B,), jnp.int32),
        pltpu.VMEM((_B, _H), jnp.float32),
        pltpu.VMEM((_B, _H), jnp.float32),
        pltpu.VMEM((_B, _H), jnp.float32),
        pltpu.VMEM((_B, _H), jnp.float32),
        pltpu.SemaphoreType.DMA,
        pltpu.SemaphoreType.DMA,
        pltpu.SemaphoreType.DMA,
        pltpu.SemaphoreType.DMA,
        pltpu.SemaphoreType.DMA,
        pltpu.SemaphoreType.DMA,
        pltpu.SemaphoreType.DMA,
        pltpu.SemaphoreType.DMA,
        pltpu.SemaphoreType.DMA,
    ],
)


_BN = 1000
_G = _N // _BN


def _tc1_body(x_ref, w_ref, dega_ref, degb_ref, y_ref):
    dis = lax.rsqrt(1.0 + (dega_ref[...] + degb_ref[...])[:, :1])
    xw = jnp.dot(x_ref[...], w_ref[...], preferred_element_type=jnp.float32)
    y_ref[...] = xw * dis


def _tc2_body(aa_ref, ab_ref, ya_ref, yb_ref, dega_ref, degb_ref,
              b1_ref, w2_ref, y2_ref):
    dis = lax.rsqrt(1.0 + (dega_ref[...] + degb_ref[...])[:, :1])
    b1 = b1_ref[...]
    za = dis * (aa_ref[...] + ya_ref[...]) + b1[:, :_H]
    zb = dis * (ab_ref[...] + yb_ref[...]) + b1[:, _H:]
    ha = jnp.where(za >= 0, za, 0.01 * za)
    hb = jnp.where(zb >= 0, zb, 0.01 * zb)
    w2 = w2_ref[...]
    y2 = (jnp.dot(ha, w2[:_H, :], preferred_element_type=jnp.float32)
          + jnp.dot(hb, w2[_H:, :], preferred_element_type=jnp.float32))
    y2_ref[...] = y2 * dis


def _tc3_body(aa_ref, ab_ref, ya_ref, yb_ref, dega_ref, degb_ref,
              b2_ref, out_ref):
    dis = lax.rsqrt(1.0 + (dega_ref[...] + degb_ref[...])[:, :1])
    oa = dis * (aa_ref[...] + ya_ref[...])
    ob = dis * (ab_ref[...] + yb_ref[...])
    out_ref[...] = jnp.concatenate([oa, ob], axis=1) + b2_ref[...]


def _tc1(x, w1, dega, degb):
    deg_spec = pl.BlockSpec((_BN, 16), lambda c, i: (i, 0))
    return pl.pallas_call(
        _tc1_body,
        grid=(2, _G),
        in_specs=[
            pl.BlockSpec((_BN, _D), lambda c, i: (i, 0)),
            pl.BlockSpec((_D, _H), lambda c, i: (0, c)),
            deg_spec, deg_spec,
        ],
        out_specs=pl.BlockSpec((_BN, _H), lambda c, i: (c * _G + i, 0)),
        out_shape=jax.ShapeDtypeStruct((2 * _N, _H), jnp.float32),
    )(x, w1, dega, degb)


def _tc2(acc1, y1, dega, degb, b1r, w2):
    half_a = pl.BlockSpec((_BN, _H), lambda c, i: (i, 0))
    half_b = pl.BlockSpec((_BN, _H), lambda c, i: (_G + i, 0))
    deg_spec = pl.BlockSpec((_BN, 16), lambda c, i: (i, 0))
    return pl.pallas_call(
        _tc2_body,
        grid=(2, _G),
        in_specs=[
            half_a, half_b, half_a, half_b,
            deg_spec, deg_spec,
            pl.BlockSpec((1, _D), lambda c, i: (0, 0)),
            pl.BlockSpec((_D, _H), lambda c, i: (0, c)),
        ],
        out_specs=pl.BlockSpec((_BN, _H), lambda c, i: (c * _G + i, 0)),
        out_shape=jax.ShapeDtypeStruct((2 * _N, _H), jnp.float32),
    )(acc1, acc1, y1, y1, dega, degb, b1r, w2)


def _tc3(acc2, y2, dega, degb, b2r):
    half_a = pl.BlockSpec((_BN, _H), lambda i: (i, 0))
    half_b = pl.BlockSpec((_BN, _H), lambda i: (_G + i, 0))
    deg_spec = pl.BlockSpec((_BN, 16), lambda i: (i, 0))
    return pl.pallas_call(
        _tc3_body,
        grid=(_G,),
        in_specs=[
            half_a, half_b, half_a, half_b,
            deg_spec, deg_spec,
            pl.BlockSpec((1, _D), lambda i: (0, 0)),
        ],
        out_specs=pl.BlockSpec((_BN, _D), lambda i: (i, 0)),
        out_shape=jax.ShapeDtypeStruct((_N, _D), jnp.float32),
    )(acc2, acc2, y2, y2, dega, degb, b2r)


def kernel(x, edge_index, edge_weight, W1, b1, W2, b2):
    src = edge_index[0]
    dst = edge_index[1]
    deg_pad = _deg(dst, edge_weight)
    dega = deg_pad[:_N]
    degb = deg_pad[_NPAD:_NPAD + _N]
    b1r = b1.reshape(1, _D)
    b2r = b2.reshape(1, _D)

    y1 = _tc1(x, W1, dega, degb)
    acc1 = _message(y1, src, dst, edge_weight)
    y2 = _tc2(acc1, y1, dega, degb, b1r, W2)
    acc2 = _message(y2, src, dst, edge_weight)
    return _tc3(acc2, y2, dega, degb, b2r)


# speedup vs baseline: 15.8459x; 1.0037x over previous
import jax
import jax.numpy as jnp
from jax import lax
from jax.experimental import pallas as pl
from jax.experimental.pallas import tpu as pltpu
from jax.experimental.pallas import tpu_sc as plsc

_N = 10000
_E = 160000
_D = 256
_H = 128
_NS = 16
_NPAD = 10240

_EPT = _E // _NS
_B = 80
_PH = 5
_PC = _EPT // _PH // _B
_DB = _E // 2 // _NS
_DBC = 128


def _sc_deg(dst_hbm, ew_hbm, deg_hbm, acc_sh, zbuf,
            ib0, ib1, eb0, eb1, dx0, dx1, pr0, pr1,
            idx8, ew16, lsem0, lsem1, scs0, scs1):
    c = lax.axis_index("c")
    s = lax.axis_index("s")
    ibs = (ib0, ib1)
    ebs = (eb0, eb1)
    dxs = (dx0, dx1)
    prs = (pr0, pr1)
    lsem = (lsem0, lsem1)
    scs = (scs0, scs1)
    base_off = c * (_E // 2) + s * _DB

    def loads_start(k, i):
        pltpu.async_copy(dst_hbm.at[pl.ds(base_off + k * _DBC, _DBC)],
                         ibs[i], lsem[i])
        pltpu.async_copy(ew_hbm.at[pl.ds(base_off + k * _DBC, _DBC)],
                         ebs[i], lsem[i])

    def loads_wait(k, i):
        pltpu.make_async_copy(dst_hbm.at[pl.ds(base_off + k * _DBC, _DBC)],
                              ibs[i], lsem[i]).wait()
        pltpu.make_async_copy(ew_hbm.at[pl.ds(base_off + k * _DBC, _DBC)],
                              ebs[i], lsem[i]).wait()

    def prep(i):
        @pl.loop(0, _DBC // 16)
        def _cp(g):
            sl = pl.ds(g * 16, 16)
            dxs[i][sl] = ibs[i][sl]

        @pl.loop(0, _DBC // 16)
        def _fill(g):
            wv = ebs[i][pl.ds(g * 16, 16)]
            for l in range(16):
                prs[i][g * 16 + l, pl.ds(0, 16)] = jnp.full((16,), wv[l],
                                                            jnp.float32)

    def scatter_start(i):
        pltpu.async_copy(prs[i], acc_sh.at[dxs[i]], scs[i], add=True)

    def scatter_wait(i):
        pltpu.make_async_copy(prs[i], acc_sh.at[dxs[i]], scs[i]).wait()

    @pl.loop(0, 640)
    def _zero(r):
        zbuf[r, pl.ds(0, 16)] = jnp.zeros((16,), jnp.float32)

    pltpu.sync_copy(zbuf, acc_sh.at[pl.ds(s * 640, 640)])
    plsc.subcore_barrier()

    loads_start(0, 0)
    loads_start(1, 1)

    @pl.loop(0, 19)
    def _pair(t):
        e = 2 * t
        o = 2 * t + 1
        loads_wait(e, 0)

        @pl.when(t >= 1)
        def _w0():
            scatter_wait(0)
        prep(0)
        loads_start(e + 2, 0)
        scatter_start(0)
        loads_wait(o, 1)

        @pl.when(t >= 1)
        def _w1():
            scatter_wait(1)
        prep(1)

        @pl.when(t < 18)
        def _l1():
            loads_start(o + 2, 1)
        scatter_start(1)

    loads_wait(38, 0)
    scatter_wait(0)
    prep(0)
    scatter_start(0)

    toff = base_off + 39 * _DBC
    pltpu.sync_copy(dst_hbm.at[pl.ds(toff, 8)], idx8)
    pltpu.sync_copy(ew_hbm.at[pl.ds(toff, 8)], ew16.at[pl.ds(0, 8)])
    scatter_wait(1)
    wv8 = ew16[pl.ds(0, 16)]
    for l in range(8):
        pr1[l, pl.ds(0, 16)] = jnp.full((16,), wv8[l], jnp.float32)
    pltpu.sync_copy(pr1.at[pl.ds(0, 8)], acc_sh.at[idx8], add=True)
    scatter_wait(0)

    plsc.subcore_barrier()
    pltpu.sync_copy(acc_sh.at[pl.ds(s * 640, 640)],
                    deg_hbm.at[pl.ds(c * _NPAD + s * 640, 640)])


def _sc_message(y_hbm, src_hbm, dst_hbm, ew_hbm, out_hbm, acc_sh,
                gidx, didxs, ews, d0, d1, d2, d3, r0, r1, r2, r3,
                g0, g1, g2, g3, s0, s1, s2, s3, stsem):
    c = lax.axis_index("c")
    s = lax.axis_index("s")
    base = c * _N
    rows = (r0, r1, r2, r3)
    d80 = (d0, d1, d2, d3)
    gsem = (g0, g1, g2, g3)
    ssem = (s0, s1, s2, s3)

    def scale(rbuf, k):
        @pl.loop(0, _B // 16)
        def _g(g):
            wv = ews[pl.ds(k * _B + g * 16, 16)]
            for l in range(16):
                wb = jnp.full((16,), wv[l], jnp.float32)
                e = g * 16 + l
                for j in range(_H // 16):
                    sl = pl.ds(j * 16, 16)
                    rbuf[e, sl] = rbuf[e, sl] * wb

    def fill(dbuf, k):
        @pl.loop(0, _B // 16)
        def _i(i):
            dbuf[pl.ds(i * 16, 16)] = didxs[pl.ds(k * _B + i * 16, 16)]

    def gather_start(k, i):
        pltpu.async_copy(y_hbm.at[gidx.at[pl.ds(k * _B, _B)]], rows[i],
                         gsem[i])

    def gather_wait(k, i):
        pltpu.make_async_copy(y_hbm.at[gidx.at[pl.ds(k * _B, _B)]],
                              rows[i], gsem[i]).wait()

    def scatter_start(i):
        pltpu.async_copy(rows[i], acc_sh.at[d80[i]], ssem[i], add=True)

    def scatter_wait(i):
        pltpu.make_async_copy(rows[i], acc_sh.at[d80[i]], ssem[i]).wait()

    @pl.loop(0, _B)
    def _zr(r):
        for j in range(_H // 16):
            r0[r, pl.ds(j * 16, 16)] = jnp.zeros((16,), jnp.float32)

    @pl.loop(0, _N // _B)
    def _zcopy(j):
        @pl.when(j % _NS == s)
        def _():
            pltpu.sync_copy(r0, acc_sh.at[pl.ds(j * _B, _B)])

    plsc.subcore_barrier()

    @pl.loop(0, _PH)
    def _phase(p):
        poff = s * _EPT + p * (_PC * _B)
        st1 = pltpu.async_copy(src_hbm.at[pl.ds(poff, _PC * _B)], gidx, stsem)
        st2 = pltpu.async_copy(dst_hbm.at[pl.ds(poff, _PC * _B)], didxs, stsem)
        st3 = pltpu.async_copy(ew_hbm.at[pl.ds(poff, _PC * _B)], ews, stsem)
        st1.wait()
        st2.wait()
        st3.wait()

        @pl.loop(0, _PC * _B // 16)
        def _adj(i):
            sl = pl.ds(i * 16, 16)
            gidx[sl] = gidx[sl] + base

        gather_start(0, 0)
        gather_start(1, 1)
        gather_start(2, 2)

        @pl.loop(0, _PC // 4)
        def _quad(t):
            k0 = 4 * t
            gather_wait(k0, 0)
            scale(r0, k0)
            fill(d0, k0)
            scatter_start(0)

            @pl.when(t >= 1)
            def _w0():
                scatter_wait(3)
            gather_start(k0 + 3, 3)
            gather_wait(k0 + 1, 1)
            scale(r1, k0 + 1)
            fill(d1, k0 + 1)
            scatter_start(1)
            scatter_wait(0)
            gather_start(k0 + 4, 0)
            gather_wait(k0 + 2, 2)
            scale(r2, k0 + 2)
            fill(d2, k0 + 2)
            scatter_start(2)

            @pl.when(t < _PC // 4 - 1)
            def _i2():
                scatter_wait(1)
                gather_start(k0 + 5, 1)
            gather_wait(k0 + 3, 3)
            scale(r3, k0 + 3)
            fill(d3, k0 + 3)
            scatter_start(3)

            @pl.when(t < _PC // 4 - 1)
            def _i3():
                scatter_wait(2)
                gather_start(k0 + 6, 2)

        last = _PC - 1
        gather_wait(last, 0)
        scale(r0, last)
        fill(d0, last)
        scatter_start(0)
        scatter_wait(1)
        scatter_wait(2)
        scatter_wait(3)
        scatter_wait(0)

    plsc.subcore_barrier()

    @pl.when(s < 10)
    def _copyout():
        pltpu.sync_copy(acc_sh.at[pl.ds(s * 1000, 1000)],
                        out_hbm.at[pl.ds(base + s * 1000, 1000)])


_MESH = plsc.VectorSubcoreMesh(core_axis_name="c", subcore_axis_name="s")

_deg = pl.kernel(
    _sc_deg,
    out_type=jax.ShapeDtypeStruct((2 * _NPAD, 16), jnp.float32),
    mesh=_MESH,
    scratch_types=[
        pltpu.VMEM_SHARED((_NPAD, 16), jnp.float32),
        pltpu.VMEM((640, 16), jnp.float32),
        pltpu.VMEM((_DBC,), jnp.int32),
        pltpu.VMEM((_DBC,), jnp.int32),
        pltpu.VMEM((_DBC,), jnp.float32),
        pltpu.VMEM((_DBC,), jnp.float32),
        pltpu.VMEM((_DBC,), jnp.int32),
        pltpu.VMEM((_DBC,), jnp.int32),
        pltpu.VMEM((_DBC, 16), jnp.float32),
        pltpu.VMEM((_DBC, 16), jnp.float32),
        pltpu.VMEM((8,), jnp.int32),
        pltpu.VMEM((16,), jnp.float32),
        pltpu.SemaphoreType.DMA,
        pltpu.SemaphoreType.DMA,
        pltpu.SemaphoreType.DMA,
        pltpu.SemaphoreType.DMA,
    ],
)

_message = pl.kernel(
    _sc_message,
    out_type=jax.ShapeDtypeStruct((2 * _N, _H), jnp.float32),
    mesh=_MESH,
    scratch_types=[
        pltpu.VMEM_SHARED((_N, _H), jnp.float32),
        pltpu.VMEM((_PC * _B,), jnp.int32),
        pltpu.VMEM((_PC * _B,), jnp.int32),
        pltpu.VMEM((_PC * _B,), jnp.float32),
        pltpu.VMEM((_B,), jnp.int32),
        pltpu.VMEM((_B,), jnp.int32),
        pltpu.VMEM((_B,), jnp.int32),
        pltpu.VMEM((_B,), jnp.int32),
        pltpu.VMEM((_B, _H), jnp.float32),
        pltpu.VMEM((_B, _H), jnp.float32),
        pltpu.VMEM((_B, _H), jnp.float32),
        pltpu.VMEM((_B, _H), jnp.float32),
        pltpu.SemaphoreType.DMA,
        pltpu.SemaphoreType.DMA,
        pltpu.SemaphoreType.DMA,
        pltpu.SemaphoreType.DMA,
        pltpu.SemaphoreType.DMA,
        pltpu.SemaphoreType.DMA,
        pltpu.SemaphoreType.DMA,
        pltpu.SemaphoreType.DMA,
        pltpu.SemaphoreType.DMA,
    ],
)


_BN = 1000
_G = _N // _BN


def _tc0_body(x_ref, w_ref, xw_ref):
    xw_ref[...] = jnp.dot(x_ref[...], w_ref[...],
                          preferred_element_type=jnp.float32)


def _tc1_body(xw_ref, dega_ref, degb_ref, y_ref):
    dis = lax.rsqrt(1.0 + (dega_ref[...] + degb_ref[...])[:, :1])
    y_ref[...] = xw_ref[...] * dis


def _tc2_body(aa_ref, ab_ref, ya_ref, yb_ref, dega_ref, degb_ref,
              b1_ref, w2_ref, y2_ref):
    dis = lax.rsqrt(1.0 + (dega_ref[...] + degb_ref[...])[:, :1])
    b1 = b1_ref[...]
    za = dis * (aa_ref[...] + ya_ref[...]) + b1[:, :_H]
    zb = dis * (ab_ref[...] + yb_ref[...]) + b1[:, _H:]
    ha = jnp.where(za >= 0, za, 0.01 * za)
    hb = jnp.where(zb >= 0, zb, 0.01 * zb)
    w2 = w2_ref[...]
    y2 = (jnp.dot(ha, w2[:_H, :], preferred_element_type=jnp.float32)
          + jnp.dot(hb, w2[_H:, :], preferred_element_type=jnp.float32))
    y2_ref[...] = y2 * dis


def _tc3_body(aa_ref, ab_ref, ya_ref, yb_ref, dega_ref, degb_ref,
              b2_ref, out_ref):
    dis = lax.rsqrt(1.0 + (dega_ref[...] + degb_ref[...])[:, :1])
    oa = dis * (aa_ref[...] + ya_ref[...])
    ob = dis * (ab_ref[...] + yb_ref[...])
    out_ref[...] = jnp.concatenate([oa, ob], axis=1) + b2_ref[...]


def _tc0(x, w1):
    return pl.pallas_call(
        _tc0_body,
        grid=(_G,),
        in_specs=[
            pl.BlockSpec((_BN, _D), lambda i: (i, 0)),
            pl.BlockSpec((_D, _D), lambda i: (0, 0)),
        ],
        out_specs=pl.BlockSpec((_BN, _D), lambda i: (i, 0)),
        out_shape=jax.ShapeDtypeStruct((_N, _D), jnp.float32),
    )(x, w1)


def _tc1(xw, dega, degb):
    deg_spec = pl.BlockSpec((_BN, 16), lambda c, i: (i, 0))
    return pl.pallas_call(
        _tc1_body,
        grid=(2, _G),
        in_specs=[
            pl.BlockSpec((_BN, _H), lambda c, i: (i, c)),
            deg_spec, deg_spec,
        ],
        out_specs=pl.BlockSpec((_BN, _H), lambda c, i: (c * _G + i, 0)),
        out_shape=jax.ShapeDtypeStruct((2 * _N, _H), jnp.float32),
    )(xw, dega, degb)


def _tc2(acc1, y1, dega, degb, b1r, w2):
    half_a = pl.BlockSpec((_BN, _H), lambda c, i: (i, 0))
    half_b = pl.BlockSpec((_BN, _H), lambda c, i: (_G + i, 0))
    deg_spec = pl.BlockSpec((_BN, 16), lambda c, i: (i, 0))
    return pl.pallas_call(
        _tc2_body,
        grid=(2, _G),
        in_specs=[
            half_a, half_b, half_a, half_b,
            deg_spec, deg_spec,
            pl.BlockSpec((1, _D), lambda c, i: (0, 0)),
            pl.BlockSpec((_D, _H), lambda c, i: (0, c)),
        ],
        out_specs=pl.BlockSpec((_BN, _H), lambda c, i: (c * _G + i, 0)),
        out_shape=jax.ShapeDtypeStruct((2 * _N, _H), jnp.float32),
    )(acc1, acc1, y1, y1, dega, degb, b1r, w2)


def _tc3(acc2, y2, dega, degb, b2r):
    half_a = pl.BlockSpec((_BN, _H), lambda i: (i, 0))
    half_b = pl.BlockSpec((_BN, _H), lambda i: (_G + i, 0))
    deg_spec = pl.BlockSpec((_BN, 16), lambda i: (i, 0))
    return pl.pallas_call(
        _tc3_body,
        grid=(_G,),
        in_specs=[
            half_a, half_b, half_a, half_b,
            deg_spec, deg_spec,
            pl.BlockSpec((1, _D), lambda i: (0, 0)),
        ],
        out_specs=pl.BlockSpec((_BN, _D), lambda i: (i, 0)),
        out_shape=jax.ShapeDtypeStruct((_N, _D), jnp.float32),
    )(acc2, acc2, y2, y2, dega, degb, b2r)


def kernel(x, edge_index, edge_weight, W1, b1, W2, b2):
    src = edge_index[0]
    dst = edge_index[1]
    xw = _tc0(x, W1)
    deg_pad = _deg(dst, edge_weight)
    dega = deg_pad[:_N]
    degb = deg_pad[_NPAD:_NPAD + _N]
    b1r = b1.reshape(1, _D)
    b2r = b2.reshape(1, _D)

    y1 = _tc1(xw, dega, degb)
    acc1 = _message(y1, src, dst, edge_weight)
    y2 = _tc2(acc1, y1, dega, degb, b1r, W2)
    acc2 = _message(y2, src, dst, edge_weight)
    return _tc3(acc2, y2, dega, degb, b2r)
